# baseline (speedup 1.0000x reference)
import jax
import jax.numpy as jnp
import numpy as np
from jax import lax
from jax.experimental import pallas as pl
from jax.experimental.pallas import tpu as pltpu
from jax.experimental.pallas import tpu_sc as plsc

_NM = 10000
_NG = 50000
_E = 640000
_D = 128
_H = 128

_NC = 2
_NS = 16
_G = 4
_GW = _H // _G
_BLK = 128
_NBT = _E // _BLK
_NBB = _NBT // _NS
_NBR = _NBT - _NBB * _NS
_NB = _NBB + 12
_RPT = 3136
_NGP = _RPT * _NS
_ZROWS = 98
_NZ = _RPT // _ZROWS
_DEG_CH = 3200
_DEG_PAD = _DEG_CH * _NS
_ZDEG = 160
_NZD = _DEG_CH // _ZDEG


def _sc_body(mesh_t, grid_t, esrc, edst, hsum_out, deg_out,
             src0, dst0, gs0, gd0,
             src1, dst1, gs1, gd1,
             src2, dst2, gs2, gd2,
             src3, dst3, gs3, gd3,
             ba0, bb0, ba1, bb1, ba2, bb2,
             bo0, bo1, dsc0, dsc1, pidx, zdeg, onesb, acc, deg_s,
             isem0, isem1, isem2, isem3,
             gsem0, gsem1, gsem2, ssem0, ssem1):
  c = lax.axis_index("c")
  s = lax.axis_index("s")
  isl = ((src0, dst0, gs0, gd0, isem0), (src1, dst1, gs1, gd1, isem1),
         (src2, dst2, gs2, gd2, isem2), (src3, dst3, gs3, gd3, isem3))
  gsl = ((ba0, bb0, gsem0), (ba1, bb1, gsem1), (ba2, bb2, gsem2))
  osl = ((bo0, dsc0, ssem0), (bo1, dsc1, ssem1))
  b0 = s * _NBB + jnp.minimum(s, _NBR)
  nreal = jnp.where(s < _NBR, _NBB + 1, _NBB)

  def _fill_zd(k, carry):
    zdeg[pl.ds(k * 16, 16)] = jnp.zeros((16,), jnp.float32)
    return carry
  lax.fori_loop(0, _ZDEG // 16, _fill_zd, 0)
  for u in range(_BLK // 16):
    onesb[pl.ds(u * 16, 16)] = jnp.full((16,), 1.0, jnp.float32)
    pidx[pl.ds(u * 16, 16)] = jnp.full((16,), _NG, jnp.int32)

  def _deg_on(body):
    @pl.when(c == 0)
    def _():
      body()

  for p in range(2):
    g = c * 2 + p

    for bo in (bo0, bo1):
      @plsc.parallel_loop(0, _BLK, 1, unroll=8)
      def _zrow(r):
        bo[r, pl.ds(0, 16)] = jnp.zeros((16,), jnp.float32)
        bo[r, pl.ds(16, 16)] = jnp.zeros((16,), jnp.float32)
    def _za(z, carry):
      pltpu.sync_copy(bo0, acc.at[pl.ds(s * _RPT + z * _BLK, _BLK)])
      return carry
    lax.fori_loop(0, _RPT // _BLK, _za, 0)
    pltpu.sync_copy(bo0.at[pl.ds(0, _RPT % _BLK)],
                    acc.at[pl.ds(s * _RPT + _RPT - _RPT % _BLK, _RPT % _BLK)])
    if p == 0:
      @_deg_on
      def _():
        def _zd(k, carry):
          pltpu.sync_copy(zdeg, deg_s.at[pl.ds(s * _DEG_CH + k * _ZDEG, _ZDEG)])
          return carry
        lax.fori_loop(0, _NZD, _zd, 0)
    plsc.subcore_barrier()

    def _stage_fire(k, it):
      src, dst, _, _, isem = isl[it]
      off = _BLK * jnp.minimum(b0 + k, _NBT - 1)
      pltpu.async_copy(esrc.at[pl.ds(off, _BLK)], src, isem)
      pltpu.async_copy(edst.at[pl.ds(off, _BLK)], dst, isem)

    def _prep_fire(k, it, gt):
      src, dst, gs, gd, isem = isl[it]
      ba, bb, gsem = gsl[gt]
      pltpu.make_async_copy(esrc.at[pl.ds(0, _BLK)], src, isem).wait()
      pltpu.make_async_copy(edst.at[pl.ds(0, _BLK)], dst, isem).wait()
      pad = k >= nreal
      for u in range(_BLK // 16):
        sl = pl.ds(u * 16, 16)
        vs = src[sl]
        vd = dst[sl]
        gs[sl] = vs * _G + g
        gd[sl] = vd * _G + g
        dst[sl] = jnp.where(pad, jnp.full((16,), _NG, jnp.int32), vd)
      pltpu.async_copy(mesh_t.at[gs], ba, gsem)
      pltpu.async_copy(grid_t.at[gd], bb, gsem)

    def _scatter_wait(ot):
      bo, dsc, ssem = osl[ot]
      pltpu.make_async_copy(bo, acc.at[pidx], ssem).wait()
      if p == 0:
        @_deg_on
        def _():
          pltpu.make_async_copy(onesb, deg_s.at[pidx], ssem).wait()

    def _crunch_scatter(it, gt, ot):
      src, dst, gs, gd, isem = isl[it]
      ba, bb, gsem = gsl[gt]
      bo, dsc, ssem = osl[ot]
      pltpu.make_async_copy(mesh_t.at[gs], ba, gsem).wait()
      pltpu.make_async_copy(grid_t.at[gd], bb, gsem).wait()
      _scatter_wait(ot)

      @plsc.parallel_loop(0, _BLK, 1, unroll=8)
      def _row(r):
        h = jnp.maximum(ba[r, pl.ds(0, _GW)] + bb[r, pl.ds(0, _GW)],
                        jnp.bfloat16(0))
        lo, hi = plsc.unpack(h, format=plsc.PackFormat.INTERLEAVED)
        bo[r, pl.ds(0, 16)] = lo
        bo[r, pl.ds(16, 16)] = hi
      for u in range(_BLK // 16):
        sl = pl.ds(u * 16, 16)
        dsc[sl] = dst[sl]
      pltpu.async_copy(bo, acc.at[dsc], ssem, add=True)
      if p == 0:
        @_deg_on
        def _():
          pltpu.async_copy(onesb, deg_s.at[dsc], ssem, add=True)

    _stage_fire(0, 0)
    _stage_fire(1, 1)
    _stage_fire(2, 2)
    _prep_fire(0, 0, 0)
    for ot in range(2):
      bo, dsc, ssem = osl[ot]
      pltpu.async_copy(bo, acc.at[pidx], ssem, add=True)
      if p == 0:
        @_deg_on
        def _():
          pltpu.async_copy(onesb, deg_s.at[pidx], ssem, add=True)

    def _pipe(kk, carry):
      for t in range(12):
        k = kk * 12 + t
        _prep_fire(k + 1, (t + 1) % 4, (t + 1) % 3)
        _stage_fire(k + 3, (t + 3) % 4)
        _crunch_scatter(t % 4, t % 3, t % 2)
      return carry
    lax.fori_loop(0, _NB // 12, _pipe, 0)

    pltpu.make_async_copy(mesh_t.at[gs0], ba0, gsem0).wait()
    pltpu.make_async_copy(grid_t.at[gd0], bb0, gsem0).wait()
    for it in (1, 2):
      src, dst, _, _, isem = isl[it]
      pltpu.make_async_copy(esrc.at[pl.ds(0, _BLK)], src, isem).wait()
      pltpu.make_async_copy(edst.at[pl.ds(0, _BLK)], dst, isem).wait()
    _scatter_wait(0)
    _scatter_wait(1)

    plsc.subcore_barrier()
    pltpu.sync_copy(acc.at[pl.ds(s * _RPT, _RPT)],
                    hsum_out.at[g, pl.ds(s * _RPT, _RPT)])
    if p == 0:
      @pl.when(c == 0)
      def _():
        pltpu.sync_copy(deg_s.at[pl.ds(s * _DEG_CH, _DEG_CH)],
                        deg_out.at[pl.ds(s * _DEG_CH, _DEG_CH)])
    plsc.subcore_barrier()


def _sc_segment(mesh_t, grid_t, esrc, edst):
  mesh = plsc.VectorSubcoreMesh(core_axis_name="c", subcore_axis_name="s")
  f32 = jnp.float32
  run = pl.kernel(
      _sc_body,
      out_type=(
          jax.ShapeDtypeStruct((_G, _NGP, _GW), f32),
          jax.ShapeDtypeStruct((_DEG_PAD,), f32),
      ),
      mesh=mesh,
      scratch_types=(
          [pltpu.VMEM((_BLK,), jnp.int32)] * 16
          + [pltpu.VMEM((_BLK, _GW), jnp.bfloat16)] * 6
          + [
              pltpu.VMEM((_BLK, _GW), f32),
              pltpu.VMEM((_BLK, _GW), f32),
              pltpu.VMEM((_BLK,), jnp.int32),
              pltpu.VMEM((_BLK,), jnp.int32),
              pltpu.VMEM((_BLK,), jnp.int32),
              pltpu.VMEM((_ZDEG,), f32),
              pltpu.VMEM((_BLK,), f32),
              pltpu.VMEM_SHARED((_NGP, _GW), f32),
              pltpu.VMEM_SHARED((_DEG_PAD,), f32),
          ]
          + [pltpu.SemaphoreType.DMA] * 9
      ),
      compiler_params=pltpu.CompilerParams(use_tc_tiling_on_sc=False,
                                           needs_layout_passes=False),
      name="mesh2grid_segment_sc",
  )
  return run(mesh_t, grid_t, esrc, edst)


_BR = 1000


def _mm_kernel(x_ref, w_ref, o_ref):
  o_ref[...] = jnp.dot(x_ref[...], w_ref[...],
                       preferred_element_type=jnp.float32
                       ).astype(o_ref.dtype)


def _mm_bias_kernel(x_ref, w_ref, b_ref, o_ref):
  o_ref[...] = (jnp.dot(x_ref[...], w_ref[...],
                        preferred_element_type=jnp.float32)
                + b_ref[...]).astype(o_ref.dtype)


def _matmul(x, w, b=None, out_dtype=jnp.float32):
  n, d = x.shape
  h = w.shape[1]
  full = lambda i: (0, 0)
  in_specs = [pl.BlockSpec((_BR, d), lambda i: (i, 0)),
              pl.BlockSpec((d, h), full)]
  args = [x, w]
  body = _mm_kernel
  if b is not None:
    in_specs.append(pl.BlockSpec((1, h), full))
    args.append(b.reshape(1, h))
    body = _mm_bias_kernel
  return pl.pallas_call(
      body,
      grid=(n // _BR,),
      in_specs=in_specs,
      out_specs=pl.BlockSpec((_BR, h), lambda i: (i, 0)),
      out_shape=jax.ShapeDtypeStruct((n, h), out_dtype),
  )(*args)


def _stageb_kernel(gx_ref, hs_ref, deg_ref, w2e_ref, b2e_ref,
                   w1g_ref, b1g_ref, w2g_ref, b2g_ref, o_ref):
  f32 = jnp.float32
  bf = lambda x: x.astype(jnp.bfloat16)
  gx = gx_ref[...]
  agg = jnp.dot(deg_ref[...], b2e_ref[...], preferred_element_type=f32)
  for g in range(_G):
    agg = agg + jnp.dot(bf(hs_ref[g]), bf(w2e_ref[g * _GW:(g + 1) * _GW, :]),
                        preferred_element_type=f32)
  pre = (jnp.dot(bf(gx), bf(w1g_ref[:_D, :]), preferred_element_type=f32)
         + jnp.dot(bf(agg), bf(w1g_ref[_D:, :]), preferred_element_type=f32)
         + b1g_ref[...])
  o_ref[...] = gx + jnp.dot(bf(jnp.maximum(pre, 0.0)), bf(w2g_ref[...]),
                            preferred_element_type=f32) + b2g_ref[...]


def _stageb(grid_x, hsum_t, deg2, w2e, b2e, w1g, b1g, w2g, b2g):
  full = lambda i: (0, 0)
  return pl.pallas_call(
      _stageb_kernel,
      grid=(_NG // _BR,),
      in_specs=[
          pl.BlockSpec((_BR, _D), lambda i: (i, 0)),
          pl.BlockSpec((_G, _BR, _GW), lambda i: (0, i, 0)),
          pl.BlockSpec((_BR, 1), lambda i: (i, 0)),
          pl.BlockSpec((_H, _H), full),
          pl.BlockSpec((1, _H), full),
          pl.BlockSpec((_D + _H, _H), full),
          pl.BlockSpec((1, _H), full),
          pl.BlockSpec((_H, _H), full),
          pl.BlockSpec((1, _H), full),
      ],
      out_specs=pl.BlockSpec((_BR, _D), lambda i: (i, 0)),
      out_shape=jax.ShapeDtypeStruct((_NG, _D), jnp.float32),
  )(grid_x, hsum_t, deg2, w2e, b2e.reshape(1, _H), w1g,
    b1g.reshape(1, _H), w2g, b2g.reshape(1, _H))


_WPERM = np.empty((_H,), dtype=np.int32)
for _g in range(_G):
  for _m in range(16):
    _WPERM[_g * _GW + 2 * _m] = _g * _GW + _m
    _WPERM[_g * _GW + 2 * _m + 1] = _g * _GW + 16 + _m


def kernel(mesh_x, grid_x, edge_src, edge_dst,
           W1e, b1e, W2e, b2e, W1g, b1g, W2g, b2g):
  w1e_p = W1e[:, _WPERM]
  b1e_p = b1e[_WPERM]
  mesh_h = _matmul(mesh_x, w1e_p[:_D], out_dtype=jnp.bfloat16)
  grid_h = _matmul(grid_x, w1e_p[_D:], b1e_p, out_dtype=jnp.bfloat16)
  mesh_t = mesh_h.reshape(_NM * _G, _GW)
  grid_t = grid_h.reshape(_NG * _G, _GW)
  hsum_t, deg_pad = _sc_segment(mesh_t, grid_t, edge_src, edge_dst)
  deg2 = deg_pad[:_NG].reshape(_NG, 1)
  return _stageb(grid_x, hsum_t, deg2, W2e, b2e, W1g, b1g, W2g, b2g)


# speedup vs baseline: 1.1091x; 1.1091x over previous
import jax
import jax.numpy as jnp
import numpy as np
from jax import lax
from jax.experimental import pallas as pl
from jax.experimental.pallas import tpu as pltpu
from jax.experimental.pallas import tpu_sc as plsc

_NM = 10000
_NG = 50000
_E = 640000
_D = 128
_H = 128

_NC = 2
_NS = 16
_G = 4
_GW = _H // _G
_BLK = 128
_NBT = _E // _BLK
_NBB = _NBT // _NS
_NBR = _NBT - _NBB * _NS
_NB = _NBB + 12
_RPT = 3136
_NGP = _RPT * _NS
_ZROWS = 98
_NZ = _RPT // _ZROWS
_DEG_CH = 3200
_DEG_PAD = _DEG_CH * _NS
_ZDEG = 160
_NZD = _DEG_CH // _ZDEG


def _sc_body(mesh_t, grid_t, esrc, edst, hsum_out, deg_out,
             src0, dst0, gs0, gd0,
             src1, dst1, gs1, gd1,
             src2, dst2, gs2, gd2,
             src3, dst3, gs3, gd3,
             ba0, bb0, ba1, bb1, ba2, bb2,
             bo0, bo1, dsc0, dsc1, pidx, zdeg, onesb, acc, deg_s,
             isem0, isem1, isem2, isem3,
             gsem0, gsem1, gsem2, ssem0, ssem1):
  c = lax.axis_index("c")
  s = lax.axis_index("s")
  isl = ((src0, dst0, gs0, gd0, isem0), (src1, dst1, gs1, gd1, isem1),
         (src2, dst2, gs2, gd2, isem2), (src3, dst3, gs3, gd3, isem3))
  gsl = ((ba0, bb0, gsem0), (ba1, bb1, gsem1), (ba2, bb2, gsem2))
  osl = ((bo0, dsc0, ssem0), (bo1, dsc1, ssem1))
  b0 = s * _NBB + jnp.minimum(s, _NBR)
  nreal = jnp.where(s < _NBR, _NBB + 1, _NBB)

  def _fill_zd(k, carry):
    zdeg[pl.ds(k * 16, 16)] = jnp.zeros((16,), jnp.float32)
    return carry
  lax.fori_loop(0, _ZDEG // 16, _fill_zd, 0)
  for u in range(_BLK // 16):
    onesb[pl.ds(u * 16, 16)] = jnp.full((16,), 1.0, jnp.float32)
    pidx[pl.ds(u * 16, 16)] = jnp.full((16,), _NG, jnp.int32)

  def _deg_on(body):
    @pl.when(c == 0)
    def _():
      body()

  for p in range(2):
    g = c * 2 + p

    for bo in (bo0, bo1):
      @plsc.parallel_loop(0, _BLK, 1, unroll=8)
      def _zrow(r):
        bo[r, pl.ds(0, 16)] = jnp.zeros((16,), jnp.float32)
        bo[r, pl.ds(16, 16)] = jnp.zeros((16,), jnp.float32)
    def _za(z, carry):
      pltpu.sync_copy(bo0, acc.at[pl.ds(s * _RPT + z * _BLK, _BLK)])
      return carry
    lax.fori_loop(0, _RPT // _BLK, _za, 0)
    pltpu.sync_copy(bo0.at[pl.ds(0, _RPT % _BLK)],
                    acc.at[pl.ds(s * _RPT + _RPT - _RPT % _BLK, _RPT % _BLK)])
    if p == 0:
      @_deg_on
      def _():
        def _zd(k, carry):
          pltpu.sync_copy(zdeg, deg_s.at[pl.ds(s * _DEG_CH + k * _ZDEG, _ZDEG)])
          return carry
        lax.fori_loop(0, _NZD, _zd, 0)
    plsc.subcore_barrier()

    def _stage_fire(k, it):
      src, dst, _, _, isem = isl[it]
      off = _BLK * jnp.minimum(b0 + k, _NBT - 1)
      pltpu.async_copy(esrc.at[pl.ds(off, _BLK)], src, isem)
      pltpu.async_copy(edst.at[pl.ds(off, _BLK)], dst, isem)

    def _prep_fire(k, it, gt):
      src, dst, gs, gd, isem = isl[it]
      ba, bb, gsem = gsl[gt]
      pltpu.make_async_copy(esrc.at[pl.ds(0, _BLK)], src, isem).wait()
      pltpu.make_async_copy(edst.at[pl.ds(0, _BLK)], dst, isem).wait()
      pad = k >= nreal
      for u in range(_BLK // 16):
        sl = pl.ds(u * 16, 16)
        vs = src[sl]
        vd = dst[sl]
        gs[sl] = vs * _G + g
        gd[sl] = vd * _G + g
        dst[sl] = jnp.where(pad, jnp.full((16,), _NG, jnp.int32), vd)
      pltpu.async_copy(mesh_t.at[gs], ba, gsem)
      pltpu.async_copy(grid_t.at[gd], bb, gsem)

    def _scatter_wait(ot):
      bo, dsc, ssem = osl[ot]
      pltpu.make_async_copy(bo, acc.at[pidx], ssem).wait()
      if p == 0:
        @_deg_on
        def _():
          pltpu.make_async_copy(onesb, deg_s.at[pidx], ssem).wait()

    def _crunch_scatter(it, gt, ot):
      src, dst, gs, gd, isem = isl[it]
      ba, bb, gsem = gsl[gt]
      bo, dsc, ssem = osl[ot]
      pltpu.make_async_copy(mesh_t.at[gs], ba, gsem).wait()
      pltpu.make_async_copy(grid_t.at[gd], bb, gsem).wait()
      _scatter_wait(ot)

      @plsc.parallel_loop(0, _BLK, 1, unroll=16)
      def _row(r):
        h = jnp.maximum(ba[r, pl.ds(0, _GW)] + bb[r, pl.ds(0, _GW)],
                        jnp.bfloat16(0))
        lo, hi = plsc.unpack(h, format=plsc.PackFormat.INTERLEAVED)
        bo[r, pl.ds(0, 16)] = lo
        bo[r, pl.ds(16, 16)] = hi
      for u in range(_BLK // 16):
        sl = pl.ds(u * 16, 16)
        dsc[sl] = dst[sl]
      pltpu.async_copy(bo, acc.at[dsc], ssem, add=True)
      if p == 0:
        @_deg_on
        def _():
          pltpu.async_copy(onesb, deg_s.at[dsc], ssem, add=True)

    _stage_fire(0, 0)
    _stage_fire(1, 1)
    _stage_fire(2, 2)
    _prep_fire(0, 0, 0)
    for ot in range(2):
      bo, dsc, ssem = osl[ot]
      pltpu.async_copy(bo, acc.at[pidx], ssem, add=True)
      if p == 0:
        @_deg_on
        def _():
          pltpu.async_copy(onesb, deg_s.at[pidx], ssem, add=True)

    def _pipe(kk, carry):
      for t in range(12):
        k = kk * 12 + t
        _prep_fire(k + 1, (t + 1) % 4, (t + 1) % 3)
        _stage_fire(k + 3, (t + 3) % 4)
        _crunch_scatter(t % 4, t % 3, t % 2)
      return carry
    lax.fori_loop(0, _NB // 12, _pipe, 0)

    pltpu.make_async_copy(mesh_t.at[gs0], ba0, gsem0).wait()
    pltpu.make_async_copy(grid_t.at[gd0], bb0, gsem0).wait()
    for it in (1, 2):
      src, dst, _, _, isem = isl[it]
      pltpu.make_async_copy(esrc.at[pl.ds(0, _BLK)], src, isem).wait()
      pltpu.make_async_copy(edst.at[pl.ds(0, _BLK)], dst, isem).wait()
    _scatter_wait(0)
    _scatter_wait(1)

    plsc.subcore_barrier()
    pltpu.sync_copy(acc.at[pl.ds(s * _RPT, _RPT)],
                    hsum_out.at[g, pl.ds(s * _RPT, _RPT)])
    if p == 0:
      @pl.when(c == 0)
      def _():
        pltpu.sync_copy(deg_s.at[pl.ds(s * _DEG_CH, _DEG_CH)],
                        deg_out.at[pl.ds(s * _DEG_CH, _DEG_CH)])
    plsc.subcore_barrier()


def _sc_segment(mesh_t, grid_t, esrc, edst):
  mesh = plsc.VectorSubcoreMesh(core_axis_name="c", subcore_axis_name="s")
  f32 = jnp.float32
  run = pl.kernel(
      _sc_body,
      out_type=(
          jax.ShapeDtypeStruct((_G, _NGP, _GW), f32),
          jax.ShapeDtypeStruct((_DEG_PAD,), f32),
      ),
      mesh=mesh,
      scratch_types=(
          [pltpu.VMEM((_BLK,), jnp.int32)] * 16
          + [pltpu.VMEM((_BLK, _GW), jnp.bfloat16)] * 6
          + [
              pltpu.VMEM((_BLK, _GW), f32),
              pltpu.VMEM((_BLK, _GW), f32),
              pltpu.VMEM((_BLK,), jnp.int32),
              pltpu.VMEM((_BLK,), jnp.int32),
              pltpu.VMEM((_BLK,), jnp.int32),
              pltpu.VMEM((_ZDEG,), f32),
              pltpu.VMEM((_BLK,), f32),
              pltpu.VMEM_SHARED((_NGP, _GW), f32),
              pltpu.VMEM_SHARED((_DEG_PAD,), f32),
          ]
          + [pltpu.SemaphoreType.DMA] * 9
      ),
      compiler_params=pltpu.CompilerParams(use_tc_tiling_on_sc=False,
                                           needs_layout_passes=False),
      name="mesh2grid_segment_sc",
  )
  return run(mesh_t, grid_t, esrc, edst)


_BR = 1000


def _mm_kernel(x_ref, w_ref, o_ref):
  o_ref[...] = jnp.dot(x_ref[...], w_ref[...],
                       preferred_element_type=jnp.float32
                       ).astype(o_ref.dtype)


def _mm_bias_kernel(x_ref, w_ref, b_ref, o_ref):
  o_ref[...] = (jnp.dot(x_ref[...], w_ref[...],
                        preferred_element_type=jnp.float32)
                + b_ref[...]).astype(o_ref.dtype)


def _matmul(x, w, b=None, out_dtype=jnp.float32):
  n, d = x.shape
  h = w.shape[1]
  full = lambda i: (0, 0)
  in_specs = [pl.BlockSpec((_BR, d), lambda i: (i, 0)),
              pl.BlockSpec((d, h), full)]
  args = [x, w]
  body = _mm_kernel
  if b is not None:
    in_specs.append(pl.BlockSpec((1, h), full))
    args.append(b.reshape(1, h))
    body = _mm_bias_kernel
  return pl.pallas_call(
      body,
      grid=(n // _BR,),
      in_specs=in_specs,
      out_specs=pl.BlockSpec((_BR, h), lambda i: (i, 0)),
      out_shape=jax.ShapeDtypeStruct((n, h), out_dtype),
  )(*args)


def _stageb_kernel(gx_ref, hs_ref, deg_ref, w2e_ref, b2e_ref,
                   w1g_ref, b1g_ref, w2g_ref, b2g_ref, o_ref):
  f32 = jnp.float32
  bf = lambda x: x.astype(jnp.bfloat16)
  gx = gx_ref[...]
  agg = jnp.dot(deg_ref[...], b2e_ref[...], preferred_element_type=f32)
  for g in range(_G):
    agg = agg + jnp.dot(bf(hs_ref[g]), bf(w2e_ref[g * _GW:(g + 1) * _GW, :]),
                        preferred_element_type=f32)
  pre = (jnp.dot(bf(gx), bf(w1g_ref[:_D, :]), preferred_element_type=f32)
         + jnp.dot(bf(agg), bf(w1g_ref[_D:, :]), preferred_element_type=f32)
         + b1g_ref[...])
  o_ref[...] = gx + jnp.dot(bf(jnp.maximum(pre, 0.0)), bf(w2g_ref[...]),
                            preferred_element_type=f32) + b2g_ref[...]


def _stageb(grid_x, hsum_t, deg2, w2e, b2e, w1g, b1g, w2g, b2g):
  full = lambda i: (0, 0)
  return pl.pallas_call(
      _stageb_kernel,
      grid=(_NG // _BR,),
      in_specs=[
          pl.BlockSpec((_BR, _D), lambda i: (i, 0)),
          pl.BlockSpec((_G, _BR, _GW), lambda i: (0, i, 0)),
          pl.BlockSpec((_BR, 1), lambda i: (i, 0)),
          pl.BlockSpec((_H, _H), full),
          pl.BlockSpec((1, _H), full),
          pl.BlockSpec((_D + _H, _H), full),
          pl.BlockSpec((1, _H), full),
          pl.BlockSpec((_H, _H), full),
          pl.BlockSpec((1, _H), full),
      ],
      out_specs=pl.BlockSpec((_BR, _D), lambda i: (i, 0)),
      out_shape=jax.ShapeDtypeStruct((_NG, _D), jnp.float32),
  )(grid_x, hsum_t, deg2, w2e, b2e.reshape(1, _H), w1g,
    b1g.reshape(1, _H), w2g, b2g.reshape(1, _H))


_WPERM = np.empty((_H,), dtype=np.int32)
for _g in range(_G):
  for _m in range(16):
    _WPERM[_g * _GW + 2 * _m] = _g * _GW + _m
    _WPERM[_g * _GW + 2 * _m + 1] = _g * _GW + 16 + _m


def kernel(mesh_x, grid_x, edge_src, edge_dst,
           W1e, b1e, W2e, b2e, W1g, b1g, W2g, b2g):
  w1e_p = W1e[:, _WPERM]
  b1e_p = b1e[_WPERM]
  mesh_h = _matmul(mesh_x, w1e_p[:_D], out_dtype=jnp.bfloat16)
  grid_h = _matmul(grid_x, w1e_p[_D:], b1e_p, out_dtype=jnp.bfloat16)
  mesh_t = mesh_h.reshape(_NM * _G, _GW)
  grid_t = grid_h.reshape(_NG * _G, _GW)
  hsum_t, deg_pad = _sc_segment(mesh_t, grid_t, edge_src, edge_dst)
  deg2 = deg_pad[:_NG].reshape(_NG, 1)
  return _stageb(grid_x, hsum_t, deg2, W2e, b2e, W1g, b1g, W2g, b2g)
